# baseline (device time: 81492 ns/iter reference)
import jax
import jax.numpy as jnp
from jax import lax
from jax.experimental import pallas as pl
from jax.experimental.pallas import tpu as pltpu

N_DEV = 4


def kernel(x, w_mat, scale_x, scale_w):
    m_per, k = x.shape
    _, n_per = w_mat.shape

    A = (0, 128)
    B = (128, 256)
    C = (384, 256)
    E = (640, 256)
    F = (896, 128)
    R2 = (384, 640)
    L2 = (0, 640)
    FR1 = (0, 384)
    FR2 = (384, 128)
    FL1 = (640, 384)
    FL2 = (512, 128)

    def ds(r):
        return pl.ds(r[0], r[1])

    def body(x_hbm, w_hbm, sx_ref, sw_ref, out_hbm,
             xv, wv, comm, acc, snd, rcv, lsem, osem):
        my = lax.axis_index("i")
        left = lax.rem(my + (N_DEV - 1), N_DEV)
        right = lax.rem(my + 1, N_DEV)

        def load(r, i):
            cp = pltpu.make_async_copy(
                x_hbm.at[ds(r), :], xv.at[ds(r), :], lsem.at[i])
            cp.start()
            return cp

        cp_a = load(A, 0)
        cp_f = load(F, 1)

        barrier_sem = pltpu.get_barrier_semaphore()
        for nbr in (left, right):
            pl.semaphore_signal(
                barrier_sem, inc=1,
                device_id=(nbr,), device_id_type=pl.DeviceIdType.MESH,
            )
        pl.semaphore_wait(barrier_sem, 2)

        def rc(src_slot, dst_slot, r, i, tgt):
            r_ = pltpu.make_async_remote_copy(
                src_ref=comm.at[src_slot, ds(r)],
                dst_ref=comm.at[dst_slot, ds(r)],
                send_sem=snd.at[i], recv_sem=rcv.at[i],
                device_id=(tgt,), device_id_type=pl.DeviceIdType.MESH,
            )
            r_.start()
            return r_

        def cast(r):
            comm[0, ds(r)] = xv[ds(r), :].astype(jnp.float8_e5m2)

        cp_a.wait()
        cast(A)
        h1r_a = rc(0, 1, A, 0, right)
        cp_f.wait()
        cast(F)
        h1l_f = rc(0, 2, F, 3, left)
        cp_b = load(B, 2)
        cp_e = load(E, 3)
        cp_c = load(C, 4)
        w_cp = pltpu.make_async_copy(w_hbm, wv, lsem.at[5])
        w_cp.start()
        cp_b.wait()
        cast(B)
        h1r_b = rc(0, 1, B, 1, right)
        cp_e.wait()
        cast(E)
        h1l_e = rc(0, 2, E, 4, left)
        cp_c.wait()
        cast(C)
        h1r_2 = rc(0, 1, R2, 2, right)
        h1l_2 = rc(0, 2, L2, 5, left)

        w_cp.wait()
        w8 = wv[...].astype(jnp.float8_e5m2)
        scale = sx_ref[0] * sw_ref[0]

        out_cps = []

        def gemm_out(chunk, origin, row_off, rows, osem_i):
            y = lax.dot_general(
                chunk, w8,
                (((1,), (0,)), ((), ())),
                preferred_element_type=jnp.float32,
            )
            y = jnp.maximum(y * scale, 0.0)
            sl = pl.ds(origin * m_per + row_off, rows)
            acc[sl, :] = y
            cp = pltpu.make_async_copy(acc.at[sl, :], out_hbm.at[sl, :],
                                       osem.at[osem_i])
            cp.start()
            out_cps.append(cp)

        gemm_out(comm[0], my, 0, m_per, 0)

        h1r_a.wait_recv()
        h1r_b.wait_recv()
        h2r1 = rc(1, 3, FR1, 6, right)
        h1l_f.wait_recv()
        h1l_e.wait_recv()
        h2l1 = rc(2, 3, FL1, 8, left)
        h1r_2.wait_recv()
        h2r2 = rc(1, 3, FR2, 7, right)
        gemm_out(comm[1], left, 0, m_per, 1)
        h1l_2.wait_recv()
        h2l2 = rc(2, 3, FL2, 9, left)
        gemm_out(comm[2], right, 0, m_per, 2)

        opp = lax.rem(my + 2, N_DEV)
        h2r1.wait_recv()
        gemm_out(comm[3, ds(FR1)], opp, FR1[0], FR1[1], 3)
        h2l1.wait_recv()
        gemm_out(comm[3, ds(FL1)], opp, FL1[0], FL1[1], 4)
        h2r2.wait_recv()
        gemm_out(comm[3, ds(FR2)], opp, FR2[0], FR2[1], 5)
        h2l2.wait_recv()
        gemm_out(comm[3, ds(FL2)], opp, FL2[0], FL2[1], 6)

        for cp in out_cps:
            cp.wait()
        for r in (h1r_a, h1r_b, h1r_2, h1l_f, h1l_e, h1l_2,
                  h2r1, h2r2, h2l1, h2l2):
            r.wait_send()

    return pl.pallas_call(
        body,
        out_shape=jax.ShapeDtypeStruct((N_DEV * m_per, n_per), jnp.float32),
        in_specs=[
            pl.BlockSpec(memory_space=pl.ANY),
            pl.BlockSpec(memory_space=pl.ANY),
            pl.BlockSpec(memory_space=pltpu.SMEM),
            pl.BlockSpec(memory_space=pltpu.SMEM),
        ],
        out_specs=pl.BlockSpec(memory_space=pl.ANY),
        scratch_shapes=[
            pltpu.VMEM((m_per, k), jnp.float32),
            pltpu.VMEM((k, n_per), jnp.float32),
            pltpu.VMEM((4, m_per, k), jnp.float8_e5m2),
            pltpu.VMEM((N_DEV * m_per, n_per), jnp.float32),
            pltpu.SemaphoreType.DMA((10,)),
            pltpu.SemaphoreType.DMA((10,)),
            pltpu.SemaphoreType.DMA((6,)),
            pltpu.SemaphoreType.DMA((7,)),
        ],
        compiler_params=pltpu.CompilerParams(
            collective_id=0,
            vmem_limit_bytes=100 * 1024 * 1024,
        ),
    )(x, w_mat, scale_x, scale_w)


# device time: 73535 ns/iter; 1.1082x vs baseline; 1.1082x over previous
import jax
import jax.numpy as jnp
from jax import lax
from jax.experimental import pallas as pl
from jax.experimental.pallas import tpu as pltpu

N_DEV = 4


def kernel(x, w_mat, scale_x, scale_w):
    m_per, k = x.shape
    _, n_per = w_mat.shape
    k_half = k // 2
    m_half = m_per // 2
    KTOP = pl.ds(0, k_half)
    KBOT = pl.ds(k_half, k_half)
    MTOP = pl.ds(0, m_half)
    MBOT = pl.ds(m_half, m_half)

    def body(x_hbm, w_hbm, sx_ref, sw_ref, out_hbm,
             xv, wv, x8, comm_w, blk_send, fr, fl, blk_recv, acc,
             snd, rcv, lsem, osem):
        my = lax.axis_index("i")
        left = lax.rem(my + (N_DEV - 1), N_DEV)
        right = lax.rem(my + 1, N_DEV)
        opp = lax.rem(my + 2, N_DEV)

        cp_wt = pltpu.make_async_copy(
            w_hbm.at[KTOP, :], wv.at[KTOP, :], lsem.at[0])
        cp_wt.start()
        cp_wb = pltpu.make_async_copy(
            w_hbm.at[KBOT, :], wv.at[KBOT, :], lsem.at[1])
        cp_wb.start()

        barrier_sem = pltpu.get_barrier_semaphore()
        for nbr in (left, right):
            pl.semaphore_signal(
                barrier_sem, inc=1,
                device_id=(nbr,), device_id_type=pl.DeviceIdType.MESH,
            )
        pl.semaphore_wait(barrier_sem, 2)

        def rc(src, dst, i, tgt):
            r_ = pltpu.make_async_remote_copy(
                src_ref=src, dst_ref=dst,
                send_sem=snd.at[i], recv_sem=rcv.at[i],
                device_id=(tgt,), device_id_type=pl.DeviceIdType.MESH,
            )
            r_.start()
            return r_

        cp_wt.wait()
        comm_w[0, KTOP, :] = wv[KTOP, :].astype(jnp.float8_e5m2)
        wr1 = rc(comm_w.at[0, KTOP], comm_w.at[1, KTOP], 0, right)
        cp_wb.wait()
        comm_w[0, KBOT, :] = wv[KBOT, :].astype(jnp.float8_e5m2)
        wr2 = rc(comm_w.at[0, KBOT], comm_w.at[1, KBOT], 1, right)
        wl1 = rc(comm_w.at[0, KBOT], comm_w.at[2, KBOT], 2, left)
        wl2 = rc(comm_w.at[0, KTOP], comm_w.at[2, KTOP], 3, left)

        cp_x = pltpu.make_async_copy(x_hbm, xv, lsem.at[2])
        cp_x.start()
        cp_x.wait()
        x8[...] = xv[...].astype(jnp.float8_e5m2)
        scale = sx_ref[0] * sw_ref[0]

        def gemm(w_chunk):
            y = lax.dot_general(
                x8[...], w_chunk,
                (((1,), (0,)), ((), ())),
                preferred_element_type=jnp.float32,
            )
            return jnp.maximum(y * scale, 0.0)

        out_cps = []

        def store_out(rows_val, origin, row_off, rows, osem_i):
            sl = pl.ds(origin * m_per + row_off, rows)
            acc[sl, :] = rows_val
            cp = pltpu.make_async_copy(acc.at[sl, :], out_hbm.at[sl, :],
                                       osem.at[osem_i])
            cp.start()
            out_cps.append(cp)

        store_out(gemm(comm_w[0]), my, 0, m_per, 0)

        wr1.wait_recv()
        fw_r = rc(comm_w.at[1, KTOP], comm_w.at[3, KTOP], 4, right)
        wl1.wait_recv()
        fw_l = rc(comm_w.at[2, KBOT], comm_w.at[3, KBOT], 5, left)

        wr2.wait_recv()
        blk_send[1] = gemm(comm_w[1]).astype(jnp.bfloat16)
        b_l = rc(blk_send.at[1], blk_recv.at[1], 7, left)
        wl2.wait_recv()
        blk_send[0] = gemm(comm_w[2]).astype(jnp.bfloat16)
        b_r = rc(blk_send.at[0], blk_recv.at[0], 6, right)

        fw_r.wait_recv()
        fw_l.wait_recv()
        blk_send[2] = gemm(comm_w[3]).astype(jnp.bfloat16)
        d_r = rc(blk_send.at[2, MTOP], fr, 8, right)
        d_l = rc(blk_send.at[2, MBOT], fl, 9, left)

        b_l_in = b_r
        b_r_in = b_l
        b_l_in.wait_recv()
        store_out(blk_recv[0].astype(jnp.float32), left, 0, m_per, 1)
        b_r_in.wait_recv()
        store_out(blk_recv[1].astype(jnp.float32), right, 0, m_per, 2)

        d_r.wait_recv()
        f_r = rc(fr, blk_recv.at[2, MTOP], 10, right)
        d_l.wait_recv()
        f_l = rc(fl, blk_recv.at[2, MBOT], 11, left)

        f_r.wait_recv()
        store_out(blk_recv[2, MTOP].astype(jnp.float32),
                  opp, 0, m_half, 3)
        f_l.wait_recv()
        store_out(blk_recv[2, MBOT].astype(jnp.float32),
                  opp, m_half, m_half, 4)

        for cp in out_cps:
            cp.wait()
        for r_ in (wr1, wr2, wl1, wl2, fw_r, fw_l,
                   b_l, b_r, d_r, d_l, f_r, f_l):
            r_.wait_send()

    return pl.pallas_call(
        body,
        out_shape=jax.ShapeDtypeStruct((N_DEV * m_per, n_per), jnp.float32),
        in_specs=[
            pl.BlockSpec(memory_space=pl.ANY),
            pl.BlockSpec(memory_space=pl.ANY),
            pl.BlockSpec(memory_space=pltpu.SMEM),
            pl.BlockSpec(memory_space=pltpu.SMEM),
        ],
        out_specs=pl.BlockSpec(memory_space=pl.ANY),
        scratch_shapes=[
            pltpu.VMEM((m_per, k), jnp.float32),
            pltpu.VMEM((k, n_per), jnp.float32),
            pltpu.VMEM((m_per, k), jnp.float8_e5m2),
            pltpu.VMEM((4, k, n_per), jnp.float8_e5m2),
            pltpu.VMEM((3, m_per, n_per), jnp.bfloat16),
            pltpu.VMEM((m_half, n_per), jnp.bfloat16),
            pltpu.VMEM((m_half, n_per), jnp.bfloat16),
            pltpu.VMEM((3, m_per, n_per), jnp.bfloat16),
            pltpu.VMEM((N_DEV * m_per, n_per), jnp.float32),
            pltpu.SemaphoreType.DMA((12,)),
            pltpu.SemaphoreType.DMA((12,)),
            pltpu.SemaphoreType.DMA((3,)),
            pltpu.SemaphoreType.DMA((5,)),
        ],
        compiler_params=pltpu.CompilerParams(
            collective_id=0,
            vmem_limit_bytes=100 * 1024 * 1024,
        ),
    )(x, w_mat, scale_x, scale_w)


# device time: 72246 ns/iter; 1.1280x vs baseline; 1.0178x over previous
import jax
import jax.numpy as jnp
from jax import lax
from jax.experimental import pallas as pl
from jax.experimental.pallas import tpu as pltpu

N_DEV = 4


def kernel(x, w_mat, scale_x, scale_w):
    m_per, k = x.shape
    _, n_per = w_mat.shape
    k_half = k // 2
    m_half = m_per // 2
    KTOP = pl.ds(0, k_half)
    KBOT = pl.ds(k_half, k_half)
    MTOP = pl.ds(0, m_half)
    MBOT = pl.ds(m_half, m_half)

    def body(x_hbm, w_hbm, sx_ref, sw_ref, out_hbm,
             xv, wv, x8, comm_w, blk_send, fr, fl, blk_recv, acc,
             snd, rcv, lsem, osem):
        my = lax.axis_index("i")
        left = lax.rem(my + (N_DEV - 1), N_DEV)
        right = lax.rem(my + 1, N_DEV)
        opp = lax.rem(my + 2, N_DEV)

        k_q = k // 4
        KQ = [pl.ds(i * k_q, k_q) for i in range(4)]

        def loadw(sl, i):
            cp = pltpu.make_async_copy(
                w_hbm.at[sl, :], wv.at[sl, :], lsem.at[i])
            cp.start()
            return cp

        cp_w0 = loadw(KQ[0], 0)
        cp_w3 = loadw(KQ[3], 3)
        cp_w1 = loadw(KQ[1], 1)
        cp_w2 = loadw(KQ[2], 2)

        barrier_sem = pltpu.get_barrier_semaphore()
        for nbr in (left, right):
            pl.semaphore_signal(
                barrier_sem, inc=1,
                device_id=(nbr,), device_id_type=pl.DeviceIdType.MESH,
            )
        pl.semaphore_wait(barrier_sem, 2)

        def rc(src, dst, i, tgt):
            r_ = pltpu.make_async_remote_copy(
                src_ref=src, dst_ref=dst,
                send_sem=snd.at[i], recv_sem=rcv.at[i],
                device_id=(tgt,), device_id_type=pl.DeviceIdType.MESH,
            )
            r_.start()
            return r_

        def castw(sl):
            comm_w[0, sl, :] = wv[sl, :].astype(jnp.float8_e5m2)

        cp_w0.wait()
        castw(KQ[0])
        wr_q0 = rc(comm_w.at[0, KQ[0]], comm_w.at[1, KQ[0]], 0, right)
        cp_w3.wait()
        castw(KQ[3])
        wl_q3 = rc(comm_w.at[0, KQ[3]], comm_w.at[2, KQ[3]], 3, left)
        cp_w1.wait()
        castw(KQ[1])
        wr_q1 = rc(comm_w.at[0, KQ[1]], comm_w.at[1, KQ[1]], 1, right)
        cp_w2.wait()
        castw(KQ[2])
        wl_q2 = rc(comm_w.at[0, KQ[2]], comm_w.at[2, KQ[2]], 4, left)
        wr_tl = rc(comm_w.at[0, KBOT], comm_w.at[1, KBOT], 2, right)
        wl_tl = rc(comm_w.at[0, KTOP], comm_w.at[2, KTOP], 5, left)

        cp_x = pltpu.make_async_copy(x_hbm, xv, lsem.at[4])
        cp_x.start()
        cp_x.wait()
        x8[...] = xv[...].astype(jnp.float8_e5m2)
        scale = sx_ref[0] * sw_ref[0]

        def gemm(w_chunk):
            y = lax.dot_general(
                x8[...], w_chunk,
                (((1,), (0,)), ((), ())),
                preferred_element_type=jnp.float32,
            )
            return jnp.maximum(y * scale, 0.0)

        out_cps = []

        def store_out(rows_val, origin, row_off, rows, osem_i):
            sl = pl.ds(origin * m_per + row_off, rows)
            acc[sl, :] = rows_val
            cp = pltpu.make_async_copy(acc.at[sl, :], out_hbm.at[sl, :],
                                       osem.at[osem_i])
            cp.start()
            out_cps.append(cp)

        store_out(gemm(comm_w[0]), my, 0, m_per, 0)

        wr_q0.wait_recv()
        wr_q1.wait_recv()
        fw_r = rc(comm_w.at[1, KTOP], comm_w.at[3, KTOP], 6, right)
        wl_q3.wait_recv()
        wl_q2.wait_recv()
        fw_l = rc(comm_w.at[2, KBOT], comm_w.at[3, KBOT], 7, left)

        wr_tl.wait_recv()
        blk_send[1] = gemm(comm_w[1]).astype(jnp.bfloat16)
        b_l = rc(blk_send.at[1], blk_recv.at[1], 9, left)
        wl_tl.wait_recv()
        blk_send[0] = gemm(comm_w[2]).astype(jnp.bfloat16)
        b_r = rc(blk_send.at[0], blk_recv.at[0], 8, right)

        fw_r.wait_recv()
        fw_l.wait_recv()
        blk_send[2] = gemm(comm_w[3]).astype(jnp.bfloat16)
        d_r = rc(blk_send.at[2, MTOP], fr, 10, right)
        d_l = rc(blk_send.at[2, MBOT], fl, 11, left)

        b_r.wait_recv()
        store_out(blk_recv[0].astype(jnp.float32), left, 0, m_per, 1)
        b_l.wait_recv()
        store_out(blk_recv[1].astype(jnp.float32), right, 0, m_per, 2)

        d_r.wait_recv()
        f_r = rc(fr, blk_recv.at[2, MTOP], 12, right)
        d_l.wait_recv()
        f_l = rc(fl, blk_recv.at[2, MBOT], 13, left)

        f_r.wait_recv()
        store_out(blk_recv[2, MTOP].astype(jnp.float32),
                  opp, 0, m_half, 3)
        f_l.wait_recv()
        store_out(blk_recv[2, MBOT].astype(jnp.float32),
                  opp, m_half, m_half, 4)

        for cp in out_cps:
            cp.wait()
        for r_ in (wr_q0, wr_q1, wr_tl, wl_q3, wl_q2, wl_tl, fw_r, fw_l,
                   b_l, b_r, d_r, d_l, f_r, f_l):
            r_.wait_send()

    return pl.pallas_call(
        body,
        out_shape=jax.ShapeDtypeStruct((N_DEV * m_per, n_per), jnp.float32),
        in_specs=[
            pl.BlockSpec(memory_space=pl.ANY),
            pl.BlockSpec(memory_space=pl.ANY),
            pl.BlockSpec(memory_space=pltpu.SMEM),
            pl.BlockSpec(memory_space=pltpu.SMEM),
        ],
        out_specs=pl.BlockSpec(memory_space=pl.ANY),
        scratch_shapes=[
            pltpu.VMEM((m_per, k), jnp.float32),
            pltpu.VMEM((k, n_per), jnp.float32),
            pltpu.VMEM((m_per, k), jnp.float8_e5m2),
            pltpu.VMEM((4, k, n_per), jnp.float8_e5m2),
            pltpu.VMEM((3, m_per, n_per), jnp.bfloat16),
            pltpu.VMEM((m_half, n_per), jnp.bfloat16),
            pltpu.VMEM((m_half, n_per), jnp.bfloat16),
            pltpu.VMEM((3, m_per, n_per), jnp.bfloat16),
            pltpu.VMEM((N_DEV * m_per, n_per), jnp.float32),
            pltpu.SemaphoreType.DMA((14,)),
            pltpu.SemaphoreType.DMA((14,)),
            pltpu.SemaphoreType.DMA((5,)),
            pltpu.SemaphoreType.DMA((5,)),
        ],
        compiler_params=pltpu.CompilerParams(
            collective_id=0,
            vmem_limit_bytes=100 * 1024 * 1024,
        ),
    )(x, w_mat, scale_x, scale_w)


# device time: 70073 ns/iter; 1.1630x vs baseline; 1.0310x over previous
import jax
import jax.numpy as jnp
from jax import lax
from jax.experimental import pallas as pl
from jax.experimental.pallas import tpu as pltpu

N_DEV = 4


def kernel(x, w_mat, scale_x, scale_w):
    m_per, k = x.shape
    _, n_per = w_mat.shape
    k_half = k // 2
    m_half = m_per // 2
    KTOP = pl.ds(0, k_half)
    KBOT = pl.ds(k_half, k_half)
    MTOP = pl.ds(0, m_half)
    MBOT = pl.ds(m_half, m_half)

    def body(x_hbm, w_hbm, sx_ref, sw_ref, out_hbm,
             xv, wv, x8, comm_w, blk_send, fr, fl, blk_recv, acc,
             snd, rcv, lsem, osem):
        my = lax.axis_index("i")
        left = lax.rem(my + (N_DEV - 1), N_DEV)
        right = lax.rem(my + 1, N_DEV)
        opp = lax.rem(my + 2, N_DEV)

        k_q = k // 4
        KQ = [pl.ds(i * k_q, k_q) for i in range(4)]

        def loadw(sl, i):
            cp = pltpu.make_async_copy(
                w_hbm.at[sl, :], wv.at[sl, :], lsem.at[i])
            cp.start()
            return cp

        cp_w0 = loadw(KQ[0], 0)
        cp_w3 = loadw(KQ[3], 3)
        cp_w1 = loadw(KQ[1], 1)
        cp_w2 = loadw(KQ[2], 2)

        barrier_sem = pltpu.get_barrier_semaphore()
        for nbr in (left, right):
            pl.semaphore_signal(
                barrier_sem, inc=1,
                device_id=(nbr,), device_id_type=pl.DeviceIdType.MESH,
            )
        pl.semaphore_wait(barrier_sem, 2)

        def rc(src, dst, i, tgt):
            r_ = pltpu.make_async_remote_copy(
                src_ref=src, dst_ref=dst,
                send_sem=snd.at[i], recv_sem=rcv.at[i],
                device_id=(tgt,), device_id_type=pl.DeviceIdType.MESH,
            )
            r_.start()
            return r_

        def castw(sl):
            comm_w[0, sl, :] = wv[sl, :].astype(jnp.float8_e5m2)

        cp_w0.wait()
        castw(KQ[0])
        wr_q0 = rc(comm_w.at[0, KQ[0]], comm_w.at[1, KQ[0]], 0, right)
        cp_w3.wait()
        castw(KQ[3])
        wl_q3 = rc(comm_w.at[0, KQ[3]], comm_w.at[2, KQ[3]], 3, left)
        cp_w1.wait()
        castw(KQ[1])
        wr_q1 = rc(comm_w.at[0, KQ[1]], comm_w.at[1, KQ[1]], 1, right)
        cp_w2.wait()
        castw(KQ[2])
        wl_q2 = rc(comm_w.at[0, KQ[2]], comm_w.at[2, KQ[2]], 4, left)
        wr_tl = rc(comm_w.at[0, KBOT], comm_w.at[1, KBOT], 2, right)
        wl_tl = rc(comm_w.at[0, KTOP], comm_w.at[2, KTOP], 5, left)

        cp_x = pltpu.make_async_copy(x_hbm, xv, lsem.at[4])
        cp_x.start()
        cp_x.wait()
        x8[...] = xv[...].astype(jnp.float8_e5m2)
        scale = sx_ref[0] * sw_ref[0]

        def gemm(w_chunk):
            y = lax.dot_general(
                x8[...], w_chunk,
                (((1,), (0,)), ((), ())),
                preferred_element_type=jnp.float32,
            )
            return jnp.maximum(y * scale, 0.0)

        out_cps = []

        def store_out(rows_val, origin, row_off, rows, osem_i):
            sl = pl.ds(origin * m_per + row_off, rows)
            acc[sl, :] = rows_val
            cp = pltpu.make_async_copy(acc.at[sl, :], out_hbm.at[sl, :],
                                       osem.at[osem_i])
            cp.start()
            out_cps.append(cp)

        store_out(gemm(comm_w[0]), my, 0, m_per, 0)

        wr_q0.wait_recv()
        wr_q1.wait_recv()
        fw_r = rc(comm_w.at[1, KTOP], comm_w.at[3, KTOP], 6, right)
        wl_q3.wait_recv()
        wl_q2.wait_recv()
        fw_l = rc(comm_w.at[2, KBOT], comm_w.at[3, KBOT], 7, left)

        wr_tl.wait_recv()
        blk_send[1] = gemm(comm_w[1]).astype(jnp.bfloat16)
        b_l = rc(blk_send.at[1], blk_recv.at[1], 9, left)
        wl_tl.wait_recv()
        blk_send[0] = gemm(comm_w[2]).astype(jnp.bfloat16)
        b_r = rc(blk_send.at[0], blk_recv.at[0], 8, right)

        m_q = m_per // 4
        MQ = [pl.ds(i * m_q, m_q) for i in range(4)]
        FQ = [pl.ds(0, m_q), pl.ds(m_q, m_q)]
        fw_r.wait_recv()
        fw_l.wait_recv()
        blk_send[2] = gemm(comm_w[3]).astype(jnp.bfloat16)
        d_r0 = rc(blk_send.at[2, MQ[0]], fr.at[FQ[0]], 10, right)
        d_r1 = rc(blk_send.at[2, MQ[1]], fr.at[FQ[1]], 11, right)
        d_l0 = rc(blk_send.at[2, MQ[3]], fl.at[FQ[1]], 12, left)
        d_l1 = rc(blk_send.at[2, MQ[2]], fl.at[FQ[0]], 13, left)

        b_r.wait_recv()
        store_out(blk_recv[0].astype(jnp.float32), left, 0, m_per, 1)
        b_l.wait_recv()
        store_out(blk_recv[1].astype(jnp.float32), right, 0, m_per, 2)

        d_r0.wait_recv()
        f_r0 = rc(fr.at[FQ[0]], blk_recv.at[2, MQ[0]], 14, right)
        d_l0.wait_recv()
        f_l0 = rc(fl.at[FQ[1]], blk_recv.at[2, MQ[3]], 16, left)
        d_r1.wait_recv()
        f_r1 = rc(fr.at[FQ[1]], blk_recv.at[2, MQ[1]], 15, right)
        d_l1.wait_recv()
        f_l1 = rc(fl.at[FQ[0]], blk_recv.at[2, MQ[2]], 17, left)

        f_r0.wait_recv()
        store_out(blk_recv[2, MQ[0]].astype(jnp.float32),
                  opp, 0, m_q, 3)
        f_l0.wait_recv()
        store_out(blk_recv[2, MQ[3]].astype(jnp.float32),
                  opp, 3 * m_q, m_q, 4)
        f_r1.wait_recv()
        store_out(blk_recv[2, MQ[1]].astype(jnp.float32),
                  opp, m_q, m_q, 5)
        f_l1.wait_recv()
        store_out(blk_recv[2, MQ[2]].astype(jnp.float32),
                  opp, 2 * m_q, m_q, 6)

        for cp in out_cps:
            cp.wait()
        for r_ in (wr_q0, wr_q1, wr_tl, wl_q3, wl_q2, wl_tl, fw_r, fw_l,
                   b_l, b_r, d_r0, d_r1, d_l0, d_l1,
                   f_r0, f_r1, f_l0, f_l1):
            r_.wait_send()

    return pl.pallas_call(
        body,
        out_shape=jax.ShapeDtypeStruct((N_DEV * m_per, n_per), jnp.float32),
        in_specs=[
            pl.BlockSpec(memory_space=pl.ANY),
            pl.BlockSpec(memory_space=pl.ANY),
            pl.BlockSpec(memory_space=pltpu.SMEM),
            pl.BlockSpec(memory_space=pltpu.SMEM),
        ],
        out_specs=pl.BlockSpec(memory_space=pl.ANY),
        scratch_shapes=[
            pltpu.VMEM((m_per, k), jnp.float32),
            pltpu.VMEM((k, n_per), jnp.float32),
            pltpu.VMEM((m_per, k), jnp.float8_e5m2),
            pltpu.VMEM((4, k, n_per), jnp.float8_e5m2),
            pltpu.VMEM((3, m_per, n_per), jnp.bfloat16),
            pltpu.VMEM((m_half, n_per), jnp.bfloat16),
            pltpu.VMEM((m_half, n_per), jnp.bfloat16),
            pltpu.VMEM((3, m_per, n_per), jnp.bfloat16),
            pltpu.VMEM((N_DEV * m_per, n_per), jnp.float32),
            pltpu.SemaphoreType.DMA((18,)),
            pltpu.SemaphoreType.DMA((18,)),
            pltpu.SemaphoreType.DMA((5,)),
            pltpu.SemaphoreType.DMA((7,)),
        ],
        compiler_params=pltpu.CompilerParams(
            collective_id=0,
            vmem_limit_bytes=100 * 1024 * 1024,
        ),
    )(x, w_mat, scale_x, scale_w)
